# baseline (speedup 1.0000x reference)
import functools

import jax
import jax.numpy as jnp
from jax import lax
from jax.experimental import pallas as pl
from jax.experimental.pallas import tpu as pltpu
from jax.experimental.pallas import tpu_sc as plsc

N_NODES = 10000
D = 128
NPAD = 10240
TRASH = N_NODES
E = 320000
NC = 2
NS = 16
NW = NC * NS
CHUNK = 128
K = 80
EPAD = NW * K * CHUNK
RPT = NPAD // NS

_sc_mesh = plsc.VectorSubcoreMesh(core_axis_name="c", subcore_axis_name="s")


GS = 4
GSUB = CHUNK // GS


def _seg_body(table, src2d, dst2d, z128,
              out_h,
              sidx, didx, rows, acc, sem):
    c = lax.axis_index("c")
    s = lax.axis_index("s")
    wid = c * NS + s
    r0 = s * RPT
    pltpu.sync_copy(z128.at[pl.ds(0, CHUNK)], rows)
    for i in range(RPT // CHUNK):
        pltpu.sync_copy(rows, acc.at[pl.ds(r0 + i * CHUNK, CHUNK)])
    pltpu.sync_copy(src2d.at[pl.ds(wid * K, K)], sidx)
    pltpu.sync_copy(dst2d.at[pl.ds(wid * K, K)], didx)
    plsc.subcore_barrier()

    @pl.loop(0, K)
    def _(j):
        for g in range(GS):
            pltpu.async_copy(table.at[sidx.at[j, pl.ds(g * GSUB, GSUB)]],
                             rows.at[pl.ds(g * GSUB, GSUB)], sem)
        for g in range(GS):
            pltpu.make_async_copy(table.at[sidx.at[j, pl.ds(g * GSUB, GSUB)]],
                                  rows.at[pl.ds(g * GSUB, GSUB)], sem).wait()
        pltpu.sync_copy(rows, acc.at[didx.at[j]], add=True)

    plsc.subcore_barrier()
    for i in range(RPT // CHUNK):
        pltpu.sync_copy(acc.at[pl.ds(r0 + i * CHUNK, CHUNK)], rows)
        pltpu.sync_copy(rows, out_h.at[c, pl.ds(r0 + i * CHUNK, CHUNK)])


def _cnt_body(d2d, z128, ones_h,
              cnt_h,
              didx, ones_v, acc):
    c = lax.axis_index("c")
    s = lax.axis_index("s")
    wid = c * NS + s
    r0 = s * RPT
    pltpu.sync_copy(z128.at[pl.ds(0, CHUNK)], ones_v)
    for i in range(RPT // CHUNK):
        pltpu.sync_copy(ones_v, acc.at[pl.ds(r0 + i * CHUNK, CHUNK)])
    pltpu.sync_copy(d2d.at[pl.ds(wid * K, K)], didx)
    pltpu.sync_copy(ones_h, ones_v)
    plsc.subcore_barrier()

    @pl.loop(0, K)
    def _(j):
        pltpu.sync_copy(ones_v, acc.at[didx.at[j]], add=True)

    plsc.subcore_barrier()
    for i in range(RPT // CHUNK):
        pltpu.sync_copy(acc.at[pl.ds(r0 + i * CHUNK, CHUNK)], ones_v)
        pltpu.sync_copy(ones_v, cnt_h.at[c, pl.ds(r0 + i * CHUNK, CHUNK)])


_seg = pl.kernel(
    _seg_body,
    out_type=jax.ShapeDtypeStruct((NC, NPAD, D), jnp.float32),
    mesh=_sc_mesh,
    scratch_types=[
        pltpu.VMEM((K, CHUNK), jnp.int32),
        pltpu.VMEM((K, CHUNK), jnp.int32),
        pltpu.VMEM((CHUNK, D), jnp.float32),
        pltpu.VMEM_SHARED((NPAD, D), jnp.float32),
        pltpu.SemaphoreType.DMA,
    ],
)

_cntk = pl.kernel(
    _cnt_body,
    out_type=jax.ShapeDtypeStruct((NC, NPAD, D), jnp.float32),
    mesh=_sc_mesh,
    scratch_types=[
        pltpu.VMEM((K, CHUNK), jnp.int32),
        pltpu.VMEM((CHUNK, D), jnp.float32),
        pltpu.VMEM_SHARED((NPAD, D), jnp.float32),
    ],
)

BT = 512


def _tc_patent_body(pC, cC, pA, cA, x, WlC, WlA, Wr, b, o):
    aggC = (pC[0] + pC[1]) / jnp.clip(cC[0, :, 0:1] + cC[1, :, 0:1], 1.0, None)
    aggA = (pA[0] + pA[1]) / jnp.clip(cA[0, :, 0:1] + cA[1, :, 0:1], 1.0, None)
    acc = jnp.dot(aggC, WlC[...], preferred_element_type=jnp.float32)
    acc = acc + jnp.dot(aggA, WlA[...], preferred_element_type=jnp.float32)
    acc = acc + jnp.dot(x[...], Wr[...], preferred_element_type=jnp.float32)
    o[...] = jnp.maximum(0.5 * (acc + b[...]), 0.0)


def _tc_patent_final_body(pC, cC, pA, cA, x, WlC, WlA, Wr, b, lW, lb, o):
    aggC = (pC[0] + pC[1]) / jnp.clip(cC[0, :, 0:1] + cC[1, :, 0:1], 1.0, None)
    aggA = (pA[0] + pA[1]) / jnp.clip(cA[0, :, 0:1] + cA[1, :, 0:1], 1.0, None)
    acc = jnp.dot(aggC, WlC[...], preferred_element_type=jnp.float32)
    acc = acc + jnp.dot(aggA, WlA[...], preferred_element_type=jnp.float32)
    acc = acc + jnp.dot(x[...], Wr[...], preferred_element_type=jnp.float32)
    h = jnp.maximum(0.5 * (acc + b[...]), 0.0)
    o[...] = jnp.dot(h, lW[...], preferred_element_type=jnp.float32) + lb[...]


def _tc_author_body(pH, cH, x, WlH, WrH, b, o):
    aggH = (pH[0] + pH[1]) / jnp.clip(cH[0, :, 0:1] + cH[1, :, 0:1], 1.0, None)
    acc = jnp.dot(aggH, WlH[...], preferred_element_type=jnp.float32)
    acc = acc + jnp.dot(x[...], WrH[...], preferred_element_type=jnp.float32)
    o[...] = jnp.maximum(acc + b[...], 0.0)


def _acc_spec():
    return pl.BlockSpec((NC, BT, D), lambda i: (0, i, 0))


def _cnt_spec():
    return pl.BlockSpec((NC, BT, D), lambda i: (0, i, 0))


def _row_spec(d=D):
    return pl.BlockSpec((BT, d), lambda i: (i, 0))


def _w_spec(r=D, c=D):
    return pl.BlockSpec((r, c), lambda i: (0, 0))


_GRID = NPAD // BT

_tc_patent = pl.pallas_call(
    _tc_patent_body,
    grid=(_GRID,),
    in_specs=[_acc_spec(), _cnt_spec(), _acc_spec(), _cnt_spec(), _row_spec(),
              _w_spec(), _w_spec(), _w_spec(), _w_spec(1, D)],
    out_specs=_row_spec(),
    out_shape=jax.ShapeDtypeStruct((NPAD, D), jnp.float32),
)

_tc_patent_final = pl.pallas_call(
    _tc_patent_final_body,
    grid=(_GRID,),
    in_specs=[_acc_spec(), _cnt_spec(), _acc_spec(), _cnt_spec(), _row_spec(),
              _w_spec(), _w_spec(), _w_spec(), _w_spec(1, D),
              _w_spec(D, 64), _w_spec(1, 64)],
    out_specs=_row_spec(64),
    out_shape=jax.ShapeDtypeStruct((NPAD, 64), jnp.float32),
)

_tc_author = pl.pallas_call(
    _tc_author_body,
    grid=(_GRID,),
    in_specs=[_acc_spec(), _cnt_spec(), _row_spec(),
              _w_spec(), _w_spec(), _w_spec(1, D)],
    out_specs=_row_spec(),
    out_shape=jax.ShapeDtypeStruct((NPAD, D), jnp.float32),
)


def _prep_edges(ei):
    src = ei[0].astype(jnp.int32)
    dst = ei[1].astype(jnp.int32)
    src = jnp.concatenate([src, jnp.zeros((EPAD - E,), jnp.int32)])
    dst = jnp.concatenate([dst, jnp.full((EPAD - E,), TRASH, jnp.int32)])
    return src.reshape(NW * K, CHUNK), dst.reshape(NW * K, CHUNK)


def _pad_rows(x):
    return jnp.concatenate([x, jnp.zeros((NPAD - N_NODES, D), x.dtype)], axis=0)


@jax.jit
def kernel(x_patent, x_author, ei_cites, ei_author_of, ei_has_author,
           c1_cites_Wl, c1_cites_bl, c1_cites_Wr,
           c1_ao_Wl, c1_ao_bl, c1_ao_Wr,
           c1_ha_Wl, c1_ha_bl, c1_ha_Wr,
           c2_cites_Wl, c2_cites_bl, c2_cites_Wr,
           c2_ao_Wl, c2_ao_bl, c2_ao_Wr,
           c2_ha_Wl, c2_ha_bl, c2_ha_Wr,
           lin_W, lin_b):
    xp = _pad_rows(x_patent)
    xa = _pad_rows(x_author)
    s_c, d_c = _prep_edges(ei_cites)
    s_a, d_a = _prep_edges(ei_author_of)
    s_h, d_h = _prep_edges(ei_has_author)
    z128 = jnp.zeros((NPAD, D), jnp.float32)
    ones_rows = jnp.ones((CHUNK, D), jnp.float32)

    cntC = _cntk(d_c, z128, ones_rows)
    cntA = _cntk(d_a, z128, ones_rows)
    cntH = _cntk(d_h, z128, ones_rows)
    accC = _seg(xp, s_c, d_c, z128)
    accA = _seg(xa, s_a, d_a, z128)
    accH = _seg(xp, s_h, d_h, z128)

    xp1 = _tc_patent(accC, cntC, accA, cntA, xp,
                     c1_cites_Wl, c1_ao_Wl, c1_cites_Wr + c1_ao_Wr,
                     (c1_cites_bl + c1_ao_bl)[None])
    xa1 = _tc_author(accH, cntH, xa, c1_ha_Wl, c1_ha_Wr, c1_ha_bl[None])

    accC2 = _seg(xp1, s_c, d_c, z128)
    accA2 = _seg(xa1, s_a, d_a, z128)

    out = _tc_patent_final(accC2, cntC, accA2, cntA, xp1,
                           c2_cites_Wl, c2_ao_Wl, c2_cites_Wr + c2_ao_Wr,
                           (c2_cites_bl + c2_ao_bl)[None],
                           lin_W, lin_b[None])
    return out[:N_NODES]


# speedup vs baseline: 1.0294x; 1.0294x over previous
import functools

import jax
import jax.numpy as jnp
from jax import lax
from jax.experimental import pallas as pl
from jax.experimental.pallas import tpu as pltpu
from jax.experimental.pallas import tpu_sc as plsc

N_NODES = 10000
D = 128
NPAD = 10240
TRASH = N_NODES
E = 320000
NC = 2
NS = 16
NW = NC * NS
CHUNK = 128
K = 80
EPAD = NW * K * CHUNK
RPT = NPAD // NS

_sc_mesh = plsc.VectorSubcoreMesh(core_axis_name="c", subcore_axis_name="s")


HW = CHUNK // 2


def _seg_body(table, src2d, dst2d, z128,
              out_h,
              sidx, didx, rows, acc, sem):
    c = lax.axis_index("c")
    s = lax.axis_index("s")
    wid = c * NS + s
    r0 = s * RPT
    pltpu.sync_copy(z128.at[pl.ds(0, CHUNK)], rows)
    for i in range(RPT // CHUNK):
        pltpu.sync_copy(rows, acc.at[pl.ds(r0 + i * CHUNK, CHUNK)])
    pltpu.sync_copy(src2d.at[pl.ds(wid * K, K)], sidx)
    pltpu.sync_copy(dst2d.at[pl.ds(wid * K, K)], didx)
    plsc.subcore_barrier()

    pltpu.async_copy(table.at[sidx.at[0, pl.ds(0, HW)]], rows.at[pl.ds(0, HW)], sem)

    @pl.loop(0, K)
    def _(j):
        for h in range(2):
            dst_half = rows.at[pl.ds(h * HW, HW)]
            pltpu.make_async_copy(table.at[sidx.at[j, pl.ds(h * HW, HW)]],
                                  dst_half, sem).wait()
            if h == 0:
                pltpu.async_copy(table.at[sidx.at[j, pl.ds(HW, HW)]],
                                 rows.at[pl.ds(HW, HW)], sem)
            else:
                @pl.when(j + 1 < K)
                def _():
                    pltpu.async_copy(table.at[sidx.at[j + 1, pl.ds(0, HW)]],
                                     rows.at[pl.ds(0, HW)], sem)

            pltpu.sync_copy(dst_half, acc.at[didx.at[j, pl.ds(h * HW, HW)]], add=True)

    plsc.subcore_barrier()
    for i in range(RPT // CHUNK):
        pltpu.sync_copy(acc.at[pl.ds(r0 + i * CHUNK, CHUNK)], rows)
        pltpu.sync_copy(rows, out_h.at[c, pl.ds(r0 + i * CHUNK, CHUNK)])


def _cnt_body(d2d, z128, ones_h,
              cnt_h,
              didx, ones_v, acc):
    c = lax.axis_index("c")
    s = lax.axis_index("s")
    wid = c * NS + s
    r0 = s * RPT
    pltpu.sync_copy(z128.at[pl.ds(0, CHUNK)], ones_v)
    for i in range(RPT // CHUNK):
        pltpu.sync_copy(ones_v, acc.at[pl.ds(r0 + i * CHUNK, CHUNK)])
    pltpu.sync_copy(d2d.at[pl.ds(wid * K, K)], didx)
    pltpu.sync_copy(ones_h, ones_v)
    plsc.subcore_barrier()

    @pl.loop(0, K)
    def _(j):
        pltpu.sync_copy(ones_v, acc.at[didx.at[j]], add=True)

    plsc.subcore_barrier()
    for i in range(RPT // CHUNK):
        pltpu.sync_copy(acc.at[pl.ds(r0 + i * CHUNK, CHUNK)], ones_v)
        pltpu.sync_copy(ones_v, cnt_h.at[c, pl.ds(r0 + i * CHUNK, CHUNK)])


_seg = pl.kernel(
    _seg_body,
    out_type=jax.ShapeDtypeStruct((NC, NPAD, D), jnp.float32),
    mesh=_sc_mesh,
    scratch_types=[
        pltpu.VMEM((K, CHUNK), jnp.int32),
        pltpu.VMEM((K, CHUNK), jnp.int32),
        pltpu.VMEM((CHUNK, D), jnp.float32),
        pltpu.VMEM_SHARED((NPAD, D), jnp.float32),
        pltpu.SemaphoreType.DMA,
    ],
)

_cntk = pl.kernel(
    _cnt_body,
    out_type=jax.ShapeDtypeStruct((NC, NPAD, D), jnp.float32),
    mesh=_sc_mesh,
    scratch_types=[
        pltpu.VMEM((K, CHUNK), jnp.int32),
        pltpu.VMEM((CHUNK, D), jnp.float32),
        pltpu.VMEM_SHARED((NPAD, D), jnp.float32),
    ],
)

BT = 512


def _tc_patent_body(pC, cC, pA, cA, x, WlC, WlA, Wr, b, o):
    aggC = (pC[0] + pC[1]) / jnp.clip(cC[0, :, 0:1] + cC[1, :, 0:1], 1.0, None)
    aggA = (pA[0] + pA[1]) / jnp.clip(cA[0, :, 0:1] + cA[1, :, 0:1], 1.0, None)
    acc = jnp.dot(aggC, WlC[...], preferred_element_type=jnp.float32)
    acc = acc + jnp.dot(aggA, WlA[...], preferred_element_type=jnp.float32)
    acc = acc + jnp.dot(x[...], Wr[...], preferred_element_type=jnp.float32)
    o[...] = jnp.maximum(0.5 * (acc + b[...]), 0.0)


def _tc_patent_final_body(pC, cC, pA, cA, x, WlC, WlA, Wr, b, lW, lb, o):
    aggC = (pC[0] + pC[1]) / jnp.clip(cC[0, :, 0:1] + cC[1, :, 0:1], 1.0, None)
    aggA = (pA[0] + pA[1]) / jnp.clip(cA[0, :, 0:1] + cA[1, :, 0:1], 1.0, None)
    acc = jnp.dot(aggC, WlC[...], preferred_element_type=jnp.float32)
    acc = acc + jnp.dot(aggA, WlA[...], preferred_element_type=jnp.float32)
    acc = acc + jnp.dot(x[...], Wr[...], preferred_element_type=jnp.float32)
    h = jnp.maximum(0.5 * (acc + b[...]), 0.0)
    o[...] = jnp.dot(h, lW[...], preferred_element_type=jnp.float32) + lb[...]


def _tc_author_body(pH, cH, x, WlH, WrH, b, o):
    aggH = (pH[0] + pH[1]) / jnp.clip(cH[0, :, 0:1] + cH[1, :, 0:1], 1.0, None)
    acc = jnp.dot(aggH, WlH[...], preferred_element_type=jnp.float32)
    acc = acc + jnp.dot(x[...], WrH[...], preferred_element_type=jnp.float32)
    o[...] = jnp.maximum(acc + b[...], 0.0)


def _acc_spec():
    return pl.BlockSpec((NC, BT, D), lambda i: (0, i, 0))


def _cnt_spec():
    return pl.BlockSpec((NC, BT, D), lambda i: (0, i, 0))


def _row_spec(d=D):
    return pl.BlockSpec((BT, d), lambda i: (i, 0))


def _w_spec(r=D, c=D):
    return pl.BlockSpec((r, c), lambda i: (0, 0))


_GRID = NPAD // BT

_tc_patent = pl.pallas_call(
    _tc_patent_body,
    grid=(_GRID,),
    in_specs=[_acc_spec(), _cnt_spec(), _acc_spec(), _cnt_spec(), _row_spec(),
              _w_spec(), _w_spec(), _w_spec(), _w_spec(1, D)],
    out_specs=_row_spec(),
    out_shape=jax.ShapeDtypeStruct((NPAD, D), jnp.float32),
)

_tc_patent_final = pl.pallas_call(
    _tc_patent_final_body,
    grid=(_GRID,),
    in_specs=[_acc_spec(), _cnt_spec(), _acc_spec(), _cnt_spec(), _row_spec(),
              _w_spec(), _w_spec(), _w_spec(), _w_spec(1, D),
              _w_spec(D, 64), _w_spec(1, 64)],
    out_specs=_row_spec(64),
    out_shape=jax.ShapeDtypeStruct((NPAD, 64), jnp.float32),
)

_tc_author = pl.pallas_call(
    _tc_author_body,
    grid=(_GRID,),
    in_specs=[_acc_spec(), _cnt_spec(), _row_spec(),
              _w_spec(), _w_spec(), _w_spec(1, D)],
    out_specs=_row_spec(),
    out_shape=jax.ShapeDtypeStruct((NPAD, D), jnp.float32),
)


def _prep_edges(ei):
    src = ei[0].astype(jnp.int32)
    dst = ei[1].astype(jnp.int32)
    src = jnp.concatenate([src, jnp.zeros((EPAD - E,), jnp.int32)])
    dst = jnp.concatenate([dst, jnp.full((EPAD - E,), TRASH, jnp.int32)])
    return src.reshape(NW * K, CHUNK), dst.reshape(NW * K, CHUNK)


def _pad_rows(x):
    return jnp.concatenate([x, jnp.zeros((NPAD - N_NODES, D), x.dtype)], axis=0)


@jax.jit
def kernel(x_patent, x_author, ei_cites, ei_author_of, ei_has_author,
           c1_cites_Wl, c1_cites_bl, c1_cites_Wr,
           c1_ao_Wl, c1_ao_bl, c1_ao_Wr,
           c1_ha_Wl, c1_ha_bl, c1_ha_Wr,
           c2_cites_Wl, c2_cites_bl, c2_cites_Wr,
           c2_ao_Wl, c2_ao_bl, c2_ao_Wr,
           c2_ha_Wl, c2_ha_bl, c2_ha_Wr,
           lin_W, lin_b):
    xp = _pad_rows(x_patent)
    xa = _pad_rows(x_author)
    s_c, d_c = _prep_edges(ei_cites)
    s_a, d_a = _prep_edges(ei_author_of)
    s_h, d_h = _prep_edges(ei_has_author)
    z128 = jnp.zeros((NPAD, D), jnp.float32)
    ones_rows = jnp.ones((CHUNK, D), jnp.float32)

    cntC = _cntk(d_c, z128, ones_rows)
    cntA = _cntk(d_a, z128, ones_rows)
    cntH = _cntk(d_h, z128, ones_rows)
    accC = _seg(xp, s_c, d_c, z128)
    accA = _seg(xa, s_a, d_a, z128)
    accH = _seg(xp, s_h, d_h, z128)

    xp1 = _tc_patent(accC, cntC, accA, cntA, xp,
                     c1_cites_Wl, c1_ao_Wl, c1_cites_Wr + c1_ao_Wr,
                     (c1_cites_bl + c1_ao_bl)[None])
    xa1 = _tc_author(accH, cntH, xa, c1_ha_Wl, c1_ha_Wr, c1_ha_bl[None])

    accC2 = _seg(xp1, s_c, d_c, z128)
    accA2 = _seg(xa1, s_a, d_a, z128)

    out = _tc_patent_final(accC2, cntC, accA2, cntA, xp1,
                           c2_cites_Wl, c2_ao_Wl, c2_cites_Wr + c2_ao_Wr,
                           (c2_cites_bl + c2_ao_bl)[None],
                           lin_W, lin_b[None])
    return out[:N_NODES]
